# baseline (speedup 1.0000x reference)
import functools

import jax
import jax.numpy as jnp
from jax import lax
from jax.experimental import pallas as pl
from jax.experimental.pallas import tpu as pltpu
from jax.experimental.pallas import tpu_sc as plsc

RESO = 32
PAD = 0.1
HID = 128
T = 16384
NSEG = RESO ** 3
NSUB = 16
CHUNK = T // NSUB
KRANGE = NSEG // NSUB


def _sort_body(p_hbm, ps_hbm, cnt_hbm,
               pbuf, keysb, rankb, posb, hist, work, extab, totals,
               prow, bcast, kpad, btab, shist, sbase, sem0):
    c = lax.axis_index("c")
    s = lax.axis_index("s")
    den = 1.0 + PAD + 1e-3
    hi = 1.0 - 1e-3

    lane = lax.iota(jnp.int32, 16)
    zi = jnp.zeros((16,), jnp.int32)
    zf = jnp.zeros((16,), jnp.float32)
    mone = jnp.full((16,), -1, jnp.int32)

    def _zp(g, _):
        prow[g, :] = zf
        return 0
    lax.fori_loop(0, CHUNK, _zp, 0)

    for bi in range(2):
        b = 2 * c + bi
        pltpu.sync_copy(p_hbm.at[b, pl.ds(s * CHUNK * 3, CHUNK * 3)], pbuf)

        def _zh(g, _):
            hist[pl.ds(g * 16, 16)] = zi
            return 0
        lax.fori_loop(0, NSEG // 16, _zh, 0)

        def _keys(g, _):
            row = g * 16 + lane
            row3 = row * 3
            x = plsc.load_gather(pbuf, [row3])
            y = plsc.load_gather(pbuf, [row3 + 1])
            z = plsc.load_gather(pbuf, [row3 + 2])
            gx = (jnp.clip(x / den + 0.5, 0.0, hi) * RESO).astype(jnp.int32)
            gy = (jnp.clip(y / den + 0.5, 0.0, hi) * RESO).astype(jnp.int32)
            gz = (jnp.clip(z / den + 0.5, 0.0, hi) * RESO).astype(jnp.int32)
            kv = gx + RESO * gy + (RESO * RESO) * gz
            keysb[pl.ds(g * 16, 16)] = kv
            plsc.store_scatter(prow, [row, zi], x)
            plsc.store_scatter(prow, [row, zi + 1], y)
            plsc.store_scatter(prow, [row, zi + 2], z)
            plsc.store_scatter(prow, [row, zi + 3],
                               plsc.bitcast(kv, jnp.float32))
            return 0
        lax.fori_loop(0, CHUNK // 16, _keys, 0)

        kpad[pl.ds(0, 16)] = mone
        kpad[pl.ds(32, 16)] = mone

        def _hist(g, _):
            kv = keysb[pl.ds(g * 16, 16)]
            kpad[pl.ds(16, 16)] = kv
            dup = jnp.zeros((16,), jnp.int32)
            later = jnp.zeros((16,), jnp.bool_)
            for d in range(1, 16):
                dup = dup + (kv == kpad[pl.ds(16 - d, 16)]).astype(jnp.int32)
                later = later | (kv == kpad[pl.ds(16 + d, 16)])
            cur = plsc.load_gather(hist, [kv])
            rankb[pl.ds(g * 16, 16)] = cur + dup
            plsc.store_scatter(hist, [kv], cur + dup + 1, mask=~later)
            return 0
        lax.fori_loop(0, CHUNK // 16, _hist, 0)

        pltpu.sync_copy(hist, shist.at[s])
        plsc.subcore_barrier()

        for t in range(NSUB):
            pltpu.sync_copy(shist.at[t, pl.ds(s * KRANGE, KRANGE)], work.at[t])

        def _cols(g, _):
            acc = jnp.zeros((16,), jnp.int32)
            for t in range(NSUB):
                v = work[t, pl.ds(g * 16, 16)]
                work[t, pl.ds(g * 16, 16)] = acc
                acc = acc + v
            totals[pl.ds(g * 16, 16)] = acc
            return 0
        lax.fori_loop(0, KRANGE // 16, _cols, 0)

        pltpu.sync_copy(totals, cnt_hbm.at[b, pl.ds(s * KRANGE, KRANGE)])

        def _scan(g, carry):
            v = totals[pl.ds(g * 16, 16)]
            cs = plsc.cumsum(v)
            extab[pl.ds(g * 16, 16)] = carry + cs - v
            return carry + jnp.sum(v)
        carry = lax.fori_loop(0, KRANGE // 16, _scan, jnp.int32(0))

        bcast[...] = jnp.full((16,), carry, jnp.int32)
        pltpu.sync_copy(bcast, sbase.at[pl.ds(s * 16, 16)])
        plsc.subcore_barrier()
        pltpu.sync_copy(sbase, btab)
        tv = plsc.load_gather(btab, [lane * 16])
        base = jnp.sum(jnp.where(lane < s, tv, 0))

        def _fix(g, _):
            e = base + extab[pl.ds(g * 16, 16)]
            for t in range(NSUB):
                work[t, pl.ds(g * 16, 16)] = work[t, pl.ds(g * 16, 16)] + e
            return 0
        lax.fori_loop(0, KRANGE // 16, _fix, 0)
        for t in range(NSUB):
            pltpu.sync_copy(work.at[t], shist.at[t, pl.ds(s * KRANGE, KRANGE)])
        plsc.subcore_barrier()

        pltpu.sync_copy(shist.at[s], hist)

        def _pos(g, _):
            kv = keysb[pl.ds(g * 16, 16)]
            sv = plsc.load_gather(hist, [kv])
            posb[pl.ds(g * 16, 16)] = sv + rankb[pl.ds(g * 16, 16)]
            return 0
        lax.fori_loop(0, CHUNK // 16, _pos, 0)

        pltpu.async_copy(prow, ps_hbm.at[b].at[posb], sem0).wait()
        plsc.subcore_barrier()


_SC_PARAMS = pltpu.CompilerParams(
    needs_layout_passes=False, use_tc_tiling_on_sc=False)


def _sc_sort(p):
    B = p.shape[0]
    pflat = p.reshape(B, T * 3)
    mesh = plsc.VectorSubcoreMesh(core_axis_name="c", subcore_axis_name="s")
    f = pl.kernel(
        _sort_body,
        out_type=(
            jax.ShapeDtypeStruct((B, T, 16), jnp.float32),
            jax.ShapeDtypeStruct((B, NSEG), jnp.int32),
        ),
        mesh=mesh,
        compiler_params=_SC_PARAMS,
        scratch_types=[
            pltpu.VMEM((CHUNK * 3,), jnp.float32),
            pltpu.VMEM((CHUNK,), jnp.int32),
            pltpu.VMEM((CHUNK,), jnp.int32),
            pltpu.VMEM((CHUNK,), jnp.int32),
            pltpu.VMEM((NSEG,), jnp.int32),
            pltpu.VMEM((NSUB, KRANGE), jnp.int32),
            pltpu.VMEM((KRANGE,), jnp.int32),
            pltpu.VMEM((KRANGE,), jnp.int32),
            pltpu.VMEM((CHUNK, 16), jnp.float32),
            pltpu.VMEM((16,), jnp.int32),
            pltpu.VMEM((48,), jnp.int32),
            pltpu.VMEM((256,), jnp.int32),
            pltpu.VMEM_SHARED((NSUB, NSEG), jnp.int32),
            pltpu.VMEM_SHARED((NSUB * 16,), jnp.int32),
            pltpu.SemaphoreType.DMA,
        ],
    )
    ps, cnt = f(pflat)
    return ps, cnt


def _block_kernel(x_ref, w0_ref, b0_ref, w1_ref, b1_ref, sc_ref, o_ref):
    x = x_ref[...]
    net = jnp.maximum(x, 0.0) @ w0_ref[...] + b0_ref[...]
    dx = jnp.maximum(net, 0.0) @ w1_ref[...] + b1_ref[...]
    o_ref[...] = x @ sc_ref[...] + dx


def _block(x, w0, b0, w1, b1, sc):
    N, K = x.shape
    TN = 2048
    b0 = b0.reshape(1, HID)
    b1 = b1.reshape(1, HID)
    return pl.pallas_call(
        _block_kernel,
        grid=(N // TN,),
        in_specs=[
            pl.BlockSpec((TN, K), lambda i: (i, 0)),
            pl.BlockSpec((K, HID), lambda i: (0, 0)),
            pl.BlockSpec((1, HID), lambda i: (0, 0)),
            pl.BlockSpec((HID, HID), lambda i: (0, 0)),
            pl.BlockSpec((1, HID), lambda i: (0, 0)),
            pl.BlockSpec((K, HID), lambda i: (0, 0)),
        ],
        out_specs=pl.BlockSpec((TN, HID), lambda i: (i, 0)),
        out_shape=jax.ShapeDtypeStruct((N, HID), jnp.float32),
    )(x, w0, b0, w1, b1, sc)


def _matmul_kernel(x_ref, w_ref, b_ref, o_ref):
    o_ref[...] = x_ref[...] @ w_ref[...] + b_ref[...]


def _matmul_bias(x, w, b):
    N, K = x.shape
    M = w.shape[1]
    TN = 2048
    b = b.reshape(1, M)
    return pl.pallas_call(
        _matmul_kernel,
        grid=(N // TN,),
        in_specs=[
            pl.BlockSpec((TN, K), lambda i: (i, 0)),
            pl.BlockSpec((K, M), lambda i: (0, 0)),
            pl.BlockSpec((1, M), lambda i: (0, 0)),
        ],
        out_specs=pl.BlockSpec((TN, M), lambda i: (i, 0)),
        out_shape=jax.ShapeDtypeStruct((N, M), jnp.float32),
    )(x, w, b)


def kernel(p, fc_pos_W, fc_pos_b, blocks_fc0_W, blocks_fc0_b, blocks_fc1_W,
           blocks_fc1_b, blocks_sc_W, fc_c_W, fc_c_b):
    B = p.shape[0]
    NB = blocks_fc0_W.shape[0]
    nseg = B * NSEG

    p_sorted, cnt = _sc_sort(p)
    skeys = lax.bitcast_convert_type(p_sorted[..., 3], jnp.int32)

    flat_idx = (skeys + jnp.arange(B, dtype=jnp.int32)[:, None] * NSEG).reshape(-1)

    w16 = jnp.zeros((16, 2 * HID), jnp.float32).at[:3].set(fc_pos_W)
    pf = p_sorted.reshape(B * T, 16)
    net = _matmul_bias(pf, w16, fc_pos_b)
    net = _block(net, blocks_fc0_W[0], blocks_fc0_b[0], blocks_fc1_W[0],
                 blocks_fc1_b[0], blocks_sc_W[0])

    for i in range(1, NB):
        seg = jax.ops.segment_max(net, flat_idx, num_segments=nseg)
        seg = jnp.where(jnp.isfinite(seg), seg, 0.0)
        pooled = seg[flat_idx]
        net = jnp.concatenate([net, pooled], axis=-1)
        net = _block(net, blocks_fc0_W[i], blocks_fc0_b[i], blocks_fc1_W[i],
                     blocks_fc1_b[i], blocks_sc_W[i])

    c = _matmul_bias(net, fc_c_W, fc_c_b)
    CDIM = c.shape[-1]

    sums = jax.ops.segment_sum(c, flat_idx, num_segments=nseg)
    cntf = cnt.reshape(-1).astype(jnp.float32)
    mean = sums / jnp.maximum(cntf, 1.0)[:, None]
    fea = mean.reshape(B, NSEG, CDIM).transpose(0, 2, 1).reshape(
        B, CDIM, RESO, RESO, RESO)

    mask = cnt.reshape(B, RESO, RESO, RESO) > 0
    return fea, mask


# speedup vs baseline: 1.5955x; 1.5955x over previous
import functools

import jax
import jax.numpy as jnp
from jax import lax
from jax.experimental import pallas as pl
from jax.experimental.pallas import tpu as pltpu
from jax.experimental.pallas import tpu_sc as plsc

RESO = 32
PAD = 0.1
HID = 128
T = 16384
NSEG = RESO ** 3
NSUB = 16
CHUNK = T // NSUB
KRANGE = NSEG // NSUB


def _sort_body(p_hbm, ps_hbm, cnt_hbm,
               pbuf, keysb, rankb, posb, hist, work, extab, totals,
               prow, bcast, kpad, btab, shist, sbase, sem0):
    c = lax.axis_index("c")
    s = lax.axis_index("s")
    den = 1.0 + PAD + 1e-3
    hi = 1.0 - 1e-3

    lane = lax.iota(jnp.int32, 16)
    zi = jnp.zeros((16,), jnp.int32)
    zf = jnp.zeros((16,), jnp.float32)
    mone = jnp.full((16,), -1, jnp.int32)

    def _zp(g, _):
        prow[g, :] = zf
        return 0
    lax.fori_loop(0, CHUNK, _zp, 0)

    for bi in range(2):
        b = 2 * c + bi
        pltpu.sync_copy(p_hbm.at[b, pl.ds(s * CHUNK * 3, CHUNK * 3)], pbuf)

        def _zh(g, _):
            hist[pl.ds(g * 16, 16)] = zi
            return 0
        lax.fori_loop(0, NSEG // 16, _zh, 0)

        def _keys(g, _):
            row = g * 16 + lane
            row3 = row * 3
            x = plsc.load_gather(pbuf, [row3])
            y = plsc.load_gather(pbuf, [row3 + 1])
            z = plsc.load_gather(pbuf, [row3 + 2])
            gx = (jnp.clip(x / den + 0.5, 0.0, hi) * RESO).astype(jnp.int32)
            gy = (jnp.clip(y / den + 0.5, 0.0, hi) * RESO).astype(jnp.int32)
            gz = (jnp.clip(z / den + 0.5, 0.0, hi) * RESO).astype(jnp.int32)
            kv = gx + RESO * gy + (RESO * RESO) * gz
            keysb[pl.ds(g * 16, 16)] = kv
            plsc.store_scatter(prow, [row, zi], x)
            plsc.store_scatter(prow, [row, zi + 1], y)
            plsc.store_scatter(prow, [row, zi + 2], z)
            plsc.store_scatter(prow, [row, zi + 3],
                               plsc.bitcast(kv, jnp.float32))
            return 0
        lax.fori_loop(0, CHUNK // 16, _keys, 0)

        kpad[pl.ds(0, 16)] = mone
        kpad[pl.ds(32, 16)] = mone

        def _hist(g, _):
            kv = keysb[pl.ds(g * 16, 16)]
            kpad[pl.ds(16, 16)] = kv
            dup = jnp.zeros((16,), jnp.int32)
            later = jnp.zeros((16,), jnp.bool_)
            for d in range(1, 16):
                dup = dup + (kv == kpad[pl.ds(16 - d, 16)]).astype(jnp.int32)
                later = later | (kv == kpad[pl.ds(16 + d, 16)])
            cur = plsc.load_gather(hist, [kv])
            rankb[pl.ds(g * 16, 16)] = cur + dup
            plsc.store_scatter(hist, [kv], cur + dup + 1, mask=~later)
            return 0
        lax.fori_loop(0, CHUNK // 16, _hist, 0)

        pltpu.sync_copy(hist, shist.at[s])
        plsc.subcore_barrier()

        for t in range(NSUB):
            pltpu.sync_copy(shist.at[t, pl.ds(s * KRANGE, KRANGE)], work.at[t])

        def _cols(g, _):
            acc = jnp.zeros((16,), jnp.int32)
            for t in range(NSUB):
                v = work[t, pl.ds(g * 16, 16)]
                work[t, pl.ds(g * 16, 16)] = acc
                acc = acc + v
            totals[pl.ds(g * 16, 16)] = acc
            return 0
        lax.fori_loop(0, KRANGE // 16, _cols, 0)

        pltpu.sync_copy(totals, cnt_hbm.at[b, pl.ds(s * KRANGE, KRANGE)])

        def _scan(g, carry):
            v = totals[pl.ds(g * 16, 16)]
            cs = plsc.cumsum(v)
            extab[pl.ds(g * 16, 16)] = carry + cs - v
            return carry + jnp.sum(v)
        carry = lax.fori_loop(0, KRANGE // 16, _scan, jnp.int32(0))

        bcast[...] = jnp.full((16,), carry, jnp.int32)
        pltpu.sync_copy(bcast, sbase.at[pl.ds(s * 16, 16)])
        plsc.subcore_barrier()
        pltpu.sync_copy(sbase, btab)
        tv = plsc.load_gather(btab, [lane * 16])
        base = jnp.sum(jnp.where(lane < s, tv, 0))

        def _fix(g, _):
            e = base + extab[pl.ds(g * 16, 16)]
            for t in range(NSUB):
                work[t, pl.ds(g * 16, 16)] = work[t, pl.ds(g * 16, 16)] + e
            return 0
        lax.fori_loop(0, KRANGE // 16, _fix, 0)
        for t in range(NSUB):
            pltpu.sync_copy(work.at[t], shist.at[t, pl.ds(s * KRANGE, KRANGE)])
        plsc.subcore_barrier()

        pltpu.sync_copy(shist.at[s], hist)

        def _pos(g, _):
            kv = keysb[pl.ds(g * 16, 16)]
            sv = plsc.load_gather(hist, [kv])
            posb[pl.ds(g * 16, 16)] = sv + rankb[pl.ds(g * 16, 16)]
            return 0
        lax.fori_loop(0, CHUNK // 16, _pos, 0)

        pltpu.async_copy(prow, ps_hbm.at[b].at[posb], sem0).wait()
        plsc.subcore_barrier()


_SC_PARAMS = pltpu.CompilerParams(
    needs_layout_passes=False, use_tc_tiling_on_sc=False)


def _sc_sort(p):
    B = p.shape[0]
    pflat = p.reshape(B, T * 3)
    mesh = plsc.VectorSubcoreMesh(core_axis_name="c", subcore_axis_name="s")
    f = pl.kernel(
        _sort_body,
        out_type=(
            jax.ShapeDtypeStruct((B, T, 16), jnp.float32),
            jax.ShapeDtypeStruct((B, NSEG), jnp.int32),
        ),
        mesh=mesh,
        compiler_params=_SC_PARAMS,
        scratch_types=[
            pltpu.VMEM((CHUNK * 3,), jnp.float32),
            pltpu.VMEM((CHUNK,), jnp.int32),
            pltpu.VMEM((CHUNK,), jnp.int32),
            pltpu.VMEM((CHUNK,), jnp.int32),
            pltpu.VMEM((NSEG,), jnp.int32),
            pltpu.VMEM((NSUB, KRANGE), jnp.int32),
            pltpu.VMEM((KRANGE,), jnp.int32),
            pltpu.VMEM((KRANGE,), jnp.int32),
            pltpu.VMEM((CHUNK, 16), jnp.float32),
            pltpu.VMEM((16,), jnp.int32),
            pltpu.VMEM((48,), jnp.int32),
            pltpu.VMEM((256,), jnp.int32),
            pltpu.VMEM_SHARED((NSUB, NSEG), jnp.int32),
            pltpu.VMEM_SHARED((NSUB * 16,), jnp.int32),
            pltpu.SemaphoreType.DMA,
        ],
    )
    ps, cnt = f(pflat)
    return ps, cnt


NSUBCH = 4
SUBR = CHUNK // NSUBCH


def _dynlane(vec, i, tmp16):
    tmp16[...] = vec
    g = plsc.load_gather(tmp16, [jnp.full((16,), i, jnp.int32)])
    return g[0]


def _edge_masks(s, myfk, mylk, mynr, fk, lk, nr, lane, tmp16, ekb):
    zi = jnp.zeros((16,), jnp.int32)

    def chain(kref):
        brk = 1 - ((nr == 1) & (fk == kref)).astype(jnp.int32)
        cs = plsc.cumsum(brk)
        cs_sm1 = _dynlane(cs, jnp.maximum(s - 1, 0), tmp16)
        cs_s = _dynlane(cs, s, tmp16)
        pred = (lane < s) & (lk == kref) & ((cs_sm1 - cs) == 0)
        ekb[pl.ds(0, 16)] = zi
        ekb[pl.ds(16, 16)] = cs
        csm1 = ekb[pl.ds(15, 16)]
        succ = (lane > s) & (fk == kref) & ((csm1 - cs_s) == 0)
        return pred, succ

    predf, succf = chain(myfk)
    predl, succl = chain(mylk)
    single = mynr == 1
    return (predf.astype(jnp.int32), (succf & single).astype(jnp.int32),
            (predl & single).astype(jnp.int32), succl.astype(jnp.int32))


def _pool_body(y_hbm, sk_hbm, out_hbm,
               ybuf, keysb, bndb, mloc, ekb, efb, elb, ekkb, tmp16,
               sedgeF, sedgeL, sedgeK):
    c = lax.axis_index("c")
    s = lax.axis_index("s")
    lane = lax.iota(jnp.int32, 16)
    NEG = jnp.float32(-3.0e38)

    def _batch(b, _):
        rbase = b * T + s * CHUNK
        pltpu.sync_copy(sk_hbm.at[b, pl.ds(s * CHUNK, CHUNK)], keysb)
        myfk = keysb[pl.ds(0, 16)][0]
        mylk = keysb[pl.ds(CHUNK - 16, 16)][15]

        rid = jnp.int32(-1)
        prevkv = jnp.full((16,), -1, jnp.int32)
        R = [jnp.full((16,), NEG, jnp.float32) for _ in range(4)]
        for sub in range(NSUBCH):
            pltpu.sync_copy(
                y_hbm.at[pl.ds(rbase + sub * SUBR, SUBR),
                         pl.ds(c * 64, 64)], ybuf)

            def _p1(g, carry):
                rid, prevkv, r0, r1, r2, r3 = carry
                R = [r0, r1, r2, r3]
                kv = keysb[pl.ds(sub * SUBR + g * 16, 16)]
                ekb[pl.ds(0, 16)] = prevkv
                ekb[pl.ds(16, 16)] = kv
                sh = ekb[pl.ds(15, 16)]
                bd = (kv != sh).astype(jnp.int32)
                bndb[pl.ds(sub * SUBR + g * 16, 16)] = bd
                for l in range(16):
                    br = bd[l] > 0
                    rid = rid + bd[l]
                    for h in range(4):
                        v = ybuf[g * 16 + l, pl.ds(h * 16, 16)]
                        R[h] = jnp.where(br, v, jnp.maximum(R[h], v))
                        mloc[rid, pl.ds(h * 16, 16)] = R[h]
                return (rid, kv, R[0], R[1], R[2], R[3])

            rid, prevkv, *R = lax.fori_loop(
                0, SUBR // 16, _p1, (rid, prevkv, *R))
        mynr = rid + 1

        pltpu.sync_copy(mloc.at[0], sedgeF.at[s])
        pltpu.sync_copy(mloc.at[mynr - 1], sedgeL.at[s])
        ekv = jnp.where(lane == 0, myfk,
                        jnp.where(lane == 1, mylk,
                                  jnp.where(lane == 2, mynr, -1)))
        ekb[pl.ds(0, 16)] = ekv
        pltpu.sync_copy(ekb.at[pl.ds(0, 16)], sedgeK.at[pl.ds(s * 16, 16)])
        plsc.subcore_barrier()

        pltpu.sync_copy(sedgeF, efb)
        pltpu.sync_copy(sedgeL, elb)
        pltpu.sync_copy(sedgeK, ekkb)
        fk = plsc.load_gather(ekkb, [lane * 16])
        lk = plsc.load_gather(ekkb, [lane * 16 + 1])
        nr = plsc.load_gather(ekkb, [lane * 16 + 2])
        fp, fs, lp, ls = _edge_masks(s, myfk, mylk, mynr, fk, lk, nr,
                                     lane, tmp16, ekb)
        for h in range(4):
            vfirst = mloc[0, pl.ds(h * 16, 16)]
            vlast = mloc[mynr - 1, pl.ds(h * 16, 16)]
            for t in range(NSUB):
                eL = elb[t, pl.ds(h * 16, 16)]
                eF = efb[t, pl.ds(h * 16, 16)]
                vfirst = jnp.where(fp[t] > 0, jnp.maximum(vfirst, eL), vfirst)
                vfirst = jnp.where(fs[t] > 0, jnp.maximum(vfirst, eF), vfirst)
                vlast = jnp.where(lp[t] > 0, jnp.maximum(vlast, eL), vlast)
                vlast = jnp.where(ls[t] > 0, jnp.maximum(vlast, eF), vlast)
            mloc[mynr - 1, pl.ds(h * 16, 16)] = vlast
            mloc[0, pl.ds(h * 16, 16)] = jnp.where(mynr == 1, vlast, vfirst)
        plsc.subcore_barrier()

        rid2 = jnp.int32(-1)
        for sub in range(NSUBCH):
            def _p2(g, rid2):
                bd = bndb[pl.ds(sub * SUBR + g * 16, 16)]
                for l in range(16):
                    rid2 = rid2 + bd[l]
                    for h in range(4):
                        ybuf[g * 16 + l, pl.ds(h * 16, 16)] = (
                            mloc[rid2, pl.ds(h * 16, 16)])
                return rid2

            rid2 = lax.fori_loop(0, SUBR // 16, _p2, rid2)
            pltpu.sync_copy(
                ybuf, out_hbm.at[pl.ds(rbase + sub * SUBR, SUBR),
                                 pl.ds(c * 64, 64)])
        return 0

    lax.fori_loop(0, 4, _batch, 0)


def _sc_pool(y, skeys):
    mesh = plsc.VectorSubcoreMesh(core_axis_name="c", subcore_axis_name="s")
    f = pl.kernel(
        _pool_body,
        out_type=jax.ShapeDtypeStruct(y.shape, jnp.float32),
        mesh=mesh,
        compiler_params=_SC_PARAMS,
        scratch_types=[
            pltpu.VMEM((SUBR, 64), jnp.float32),
            pltpu.VMEM((CHUNK,), jnp.int32),
            pltpu.VMEM((CHUNK,), jnp.int32),
            pltpu.VMEM((CHUNK, 64), jnp.float32),
            pltpu.VMEM((32,), jnp.int32),
            pltpu.VMEM((NSUB, 64), jnp.float32),
            pltpu.VMEM((NSUB, 64), jnp.float32),
            pltpu.VMEM((NSUB * 16,), jnp.int32),
            pltpu.VMEM((16,), jnp.int32),
            pltpu.VMEM_SHARED((NSUB, 64), jnp.float32),
            pltpu.VMEM_SHARED((NSUB, 64), jnp.float32),
            pltpu.VMEM_SHARED((NSUB * 16,), jnp.int32),
        ],
    )
    return f(y, skeys)


def _block_kernel(x_ref, w0_ref, b0_ref, w1_ref, b1_ref, sc_ref, o_ref):
    x = x_ref[...]
    net = jnp.maximum(x, 0.0) @ w0_ref[...] + b0_ref[...]
    dx = jnp.maximum(net, 0.0) @ w1_ref[...] + b1_ref[...]
    o_ref[...] = x @ sc_ref[...] + dx


def _block(x, w0, b0, w1, b1, sc):
    N, K = x.shape
    TN = 2048
    b0 = b0.reshape(1, HID)
    b1 = b1.reshape(1, HID)
    return pl.pallas_call(
        _block_kernel,
        grid=(N // TN,),
        in_specs=[
            pl.BlockSpec((TN, K), lambda i: (i, 0)),
            pl.BlockSpec((K, HID), lambda i: (0, 0)),
            pl.BlockSpec((1, HID), lambda i: (0, 0)),
            pl.BlockSpec((HID, HID), lambda i: (0, 0)),
            pl.BlockSpec((1, HID), lambda i: (0, 0)),
            pl.BlockSpec((K, HID), lambda i: (0, 0)),
        ],
        out_specs=pl.BlockSpec((TN, HID), lambda i: (i, 0)),
        out_shape=jax.ShapeDtypeStruct((N, HID), jnp.float32),
    )(x, w0, b0, w1, b1, sc)


def _matmul_kernel(x_ref, w_ref, b_ref, o_ref):
    o_ref[...] = x_ref[...] @ w_ref[...] + b_ref[...]


def _matmul_bias(x, w, b):
    N, K = x.shape
    M = w.shape[1]
    TN = 2048
    b = b.reshape(1, M)
    return pl.pallas_call(
        _matmul_kernel,
        grid=(N // TN,),
        in_specs=[
            pl.BlockSpec((TN, K), lambda i: (i, 0)),
            pl.BlockSpec((K, M), lambda i: (0, 0)),
            pl.BlockSpec((1, M), lambda i: (0, 0)),
        ],
        out_specs=pl.BlockSpec((TN, M), lambda i: (i, 0)),
        out_shape=jax.ShapeDtypeStruct((N, M), jnp.float32),
    )(x, w, b)


def kernel(p, fc_pos_W, fc_pos_b, blocks_fc0_W, blocks_fc0_b, blocks_fc1_W,
           blocks_fc1_b, blocks_sc_W, fc_c_W, fc_c_b):
    B = p.shape[0]
    NB = blocks_fc0_W.shape[0]
    nseg = B * NSEG

    p_sorted, cnt = _sc_sort(p)
    skeys = lax.bitcast_convert_type(p_sorted[..., 3], jnp.int32)

    w16 = jnp.zeros((16, 2 * HID), jnp.float32).at[:3].set(fc_pos_W)
    pf = p_sorted.reshape(B * T, 16)
    net = _matmul_bias(pf, w16, fc_pos_b)
    net = _block(net, blocks_fc0_W[0], blocks_fc0_b[0], blocks_fc1_W[0],
                 blocks_fc1_b[0], blocks_sc_W[0])

    for i in range(1, NB):
        pooled = _sc_pool(net, skeys)
        net = jnp.concatenate([net, pooled], axis=-1)
        net = _block(net, blocks_fc0_W[i], blocks_fc0_b[i], blocks_fc1_W[i],
                     blocks_fc1_b[i], blocks_sc_W[i])

    c = _matmul_bias(net, fc_c_W, fc_c_b)
    CDIM = c.shape[-1]

    flat_idx = (skeys + jnp.arange(B, dtype=jnp.int32)[:, None] * NSEG).reshape(-1)
    sums = jax.ops.segment_sum(c, flat_idx, num_segments=nseg)
    cntf = cnt.reshape(-1).astype(jnp.float32)
    mean = sums / jnp.maximum(cntf, 1.0)[:, None]
    fea = mean.reshape(B, NSEG, CDIM).transpose(0, 2, 1).reshape(
        B, CDIM, RESO, RESO, RESO)

    mask = cnt.reshape(B, RESO, RESO, RESO) > 0
    return fea, mask
